# baseline (speedup 1.0000x reference)
import functools

import numpy as np
import jax
import jax.numpy as jnp
from jax import lax
from jax.experimental import pallas as pl
from jax.experimental.pallas import tpu as pltpu
from jax.experimental.pallas import tpu_sc as plsc

N = 50000
E = 800000
HID = 64
H = 4
C = 16
EDIM = 7

NW = 32
SUB = 128
GROUP = 1024
EPAD = 819200
EPADH = EPAD // 2
NHALF = N // 2
ZROWS = 1568
ACC_ROWS = 16 * ZROWS
LAST_ROWS = NHALF - 15 * ZROWS

_f32 = jnp.float32

_G = np.kron(np.eye(H), np.ones((C, 1))).astype(np.float32)
_HB = np.kron(np.eye(H), np.ones((1, C))).astype(np.float32)
_P = np.eye(H, 16).astype(np.float32)
_Q = np.kron(np.eye(4, dtype=np.float32), np.ones((1, C), np.float32))
_Q = np.concatenate([_Q, np.zeros((12, 64), np.float32)], 0)

GG = 640


@functools.cache
def _gather_pair(rows):
  nper = rows // NW
  ngroups = nper // GG

  @functools.partial(
      pl.kernel,
      mesh=plsc.VectorSubcoreMesh(core_axis_name="c", subcore_axis_name="s"),
      compiler_params=pltpu.CompilerParams(use_tc_tiling_on_sc=False),
      out_type=jax.ShapeDtypeStruct((rows, 128), _f32),
      scratch_types=[
          pltpu.VMEM((2 * GG,), jnp.int32),
          pltpu.VMEM((GG, 64), _f32),
          pltpu.VMEM((GG, 64), _f32),
          pltpu.SemaphoreType.DMA,
          pltpu.SemaphoreType.DMA,
          pltpu.SemaphoreType.DMA,
      ],
  )
  def k(xl, xr, src, dst, out, idxv, bufl, bufr, seml, semr, wsem):
    wid = lax.axis_index("c") * 16 + lax.axis_index("s")
    base_w = wid * nper

    def body(g, carry):
      base = base_w + g * GG
      pltpu.sync_copy(src.at[pl.ds(base, GG)], idxv.at[pl.ds(0, GG)])
      pltpu.sync_copy(dst.at[pl.ds(base, GG)], idxv.at[pl.ds(GG, GG)])
      cpl = [
          pltpu.async_copy(xl.at[idxv.at[pl.ds(b * SUB, SUB)]],
                           bufl.at[pl.ds(b * SUB, SUB)], seml)
          for b in range(GG // SUB)
      ]
      cpr = [
          pltpu.async_copy(xr.at[idxv.at[pl.ds(GG + b * SUB, SUB)]],
                           bufr.at[pl.ds(b * SUB, SUB)], semr)
          for b in range(GG // SUB)
      ]
      for cp in cpl:
        cp.wait()
      w0 = pltpu.async_copy(bufl, out.at[pl.ds(base, GG), pl.ds(0, 64)], wsem)
      for cp in cpr:
        cp.wait()
      w1 = pltpu.async_copy(bufr, out.at[pl.ds(base, GG), pl.ds(64, 64)], wsem)
      w0.wait()
      w1.wait()
      return carry

    lax.fori_loop(0, ngroups, body, 0)

  return k


@functools.cache
def _alpha_kernel():
  nper = EPADH // NW
  ngroups = nper // GG

  @functools.partial(
      pl.kernel,
      mesh=plsc.VectorSubcoreMesh(core_axis_name="c", subcore_axis_name="s"),
      compiler_params=pltpu.CompilerParams(use_tc_tiling_on_sc=False),
      out_type=jax.ShapeDtypeStruct((EPAD, 16), _f32),
      scratch_types=[
          pltpu.VMEM((GG,), jnp.int32),
          pltpu.VMEM((GG, 16), _f32),
          pltpu.VMEM((GG, 16), _f32),
          pltpu.SemaphoreType.DMA,
      ],
  )
  def k(den, dst, paya, payb, out, idxv, dbuf, ebuf, sem):
    wid = lax.axis_index("c") * 16 + lax.axis_index("s")

    for h, pay in enumerate((paya, payb)):
      def body(g, carry, pay=pay, h=h):
        lbase = wid * nper + g * GG
        base = h * EPADH + lbase
        pltpu.sync_copy(dst.at[pl.ds(base, GG)], idxv)
        cps = [
            pltpu.async_copy(den.at[idxv.at[pl.ds(b * SUB, SUB)]],
                             dbuf.at[pl.ds(b * SUB, SUB)], sem)
            for b in range(GG // SUB)
        ]
        pltpu.sync_copy(pay.at[pl.ds(lbase, GG), pl.ds(32, 16)], ebuf)
        for cp in cps:
          cp.wait()

        def div_row(r, carry2):
          ebuf[r] = ebuf[r] / (dbuf[r] + 1e-16)
          return carry2

        lax.fori_loop(0, GG, div_row, 0)
        pltpu.sync_copy(ebuf, out.at[pl.ds(base, GG)])
        return carry

      lax.fori_loop(0, ngroups, body, 0)

  return k


ZR32 = 3126
ACC32 = 16 * ZR32
LAST32 = N - 15 * ZR32


ZD = 1563
ACCD = 16 * ZD
LASTD = NHALF - 15 * ZD


@functools.cache
def _scatter():
  group = 256
  nper = EPADH // 16
  ngroups = nper // group

  @functools.partial(
      pl.kernel,
      mesh=plsc.VectorSubcoreMesh(core_axis_name="c", subcore_axis_name="s"),
      compiler_params=pltpu.CompilerParams(use_tc_tiling_on_sc=False),
      out_type=[
          jax.ShapeDtypeStruct((N, 32), _f32),
          jax.ShapeDtypeStruct((N, 32), _f32),
          jax.ShapeDtypeStruct((N, 8), _f32),
      ],
      scratch_types=[
          pltpu.VMEM_SHARED((ACC32, 32), _f32),
          pltpu.VMEM_SHARED((ACCD, 8), _f32),
          pltpu.VMEM((group, 32), _f32),
          pltpu.VMEM((group, 8), _f32),
          pltpu.VMEM((group // SUB, SUB), jnp.int32),
          pltpu.VMEM((group // SUB, SUB), jnp.int32),
          pltpu.SemaphoreType.DMA,
      ],
  )
  def k(paya, payb, dst2d, z32, out01, out23,
        outden, acc, accd, wbuf, ebuf, idxr, idxl, sem):
    c = lax.axis_index("c")
    s = lax.axis_index("s")
    pltpu.sync_copy(z32, acc.at[pl.ds(s * ZR32, ZR32)])
    pltpu.sync_copy(z32.at[pl.ds(0, ZD), pl.ds(0, 8)],
                    accd.at[pl.ds(s * ZD, ZD)])
    plsc.subcore_barrier()

    nb = c * NHALF

    def run(col0):
      for h, pay in enumerate((paya, payb)):
        def body(g, carry, pay=pay, h=h):
          base = s * nper + g * group
          grow = (h * EPADH + s * nper + g * group) // SUB
          pltpu.sync_copy(dst2d.at[pl.ds(grow, group // SUB)], idxr)
          for r in range(group // SUB):
            for q in range(SUB // 16):
              v = idxr[r, pl.ds(q * 16, 16)]
              local = v - nb
              ok = (local >= 0) & (local < NHALF)
              dump = NHALF + (lax.iota(jnp.int32, 16) & 7)
              idxl[r, pl.ds(q * 16, 16)] = jnp.where(ok, local, dump)
          pltpu.sync_copy(pay.at[pl.ds(base, group), pl.ds(col0, 32)], wbuf)
          pltpu.sync_copy(pay.at[pl.ds(base, group), pl.ds(col0 + 32, 8)],
                          ebuf)
          for b in range(group // SUB):
            pltpu.sync_copy(wbuf.at[pl.ds(b * SUB, SUB)],
                            acc.at[idxr.at[b]], add=True)
          for b in range(group // SUB):
            pltpu.sync_copy(ebuf.at[pl.ds(b * SUB, SUB)],
                            accd.at[idxl.at[b]], add=True)
          return carry

        lax.fori_loop(0, ngroups, body, 0)

    @pl.when(c == 0)
    def _():
      run(0)

    @pl.when(c == 1)
    def _():
      run(64)

    plsc.subcore_barrier()

    def dump(out_ref):
      @pl.when(s < 15)
      def _():
        pltpu.sync_copy(acc.at[pl.ds(s * ZR32, ZR32)],
                        out_ref.at[pl.ds(s * ZR32, ZR32)])

      @pl.when(s == 15)
      def _():
        pltpu.sync_copy(acc.at[pl.ds(15 * ZR32, LAST32)],
                        out_ref.at[pl.ds(15 * ZR32, LAST32)])

    @pl.when(c == 0)
    def _():
      dump(out01)

    @pl.when(c == 1)
    def _():
      dump(out23)

    hb = c * NHALF

    @pl.when(s < 15)
    def _():
      pltpu.sync_copy(accd.at[pl.ds(s * ZD, ZD)],
                      outden.at[pl.ds(hb + s * ZD, ZD)])

    @pl.when(s == 15)
    def _():
      pltpu.sync_copy(accd.at[pl.ds(15 * ZD, LASTD)],
                      outden.at[pl.ds(hb + 15 * ZD, LASTD)])

  return k


BN = 2000
BE = 8192
BE5 = 8000


def _full(shape):
  return pl.BlockSpec(shape, lambda i: tuple(0 for _ in shape))


def _rows(block, width):
  return pl.BlockSpec((block, width), lambda i: (i, 0))


def _t1_body(x, nt, sid, temb, semb, w1, b1, w2, b2, wl, bl, wr, br,
             xl_o, xr_o):
  xx = x[...]
  oh_t = (nt[...] == lax.broadcasted_iota(jnp.int32, (BN, 2), 1)).astype(_f32)
  oh_s = (sid[...] == lax.broadcasted_iota(jnp.int32, (BN, 6), 1)).astype(_f32)
  hcat = jnp.concatenate([xx, oh_t @ temb[...], oh_s @ semb[...]], axis=1)
  h = jnp.maximum(hcat @ w1[...] + b1[...], 0.0) @ w2[...] + b2[...]
  xl_o[...] = h @ wl[...] + bl[...]
  xr_o[...] = h @ wr[...] + br[...]


def _t1(x, nt, sid, temb, semb, w1, b1, w2, b2, wl, bl, wr, br):
  return pl.pallas_call(
      _t1_body,
      grid=(N // BN,),
      in_specs=[
          _rows(BN, 7), _rows(BN, 1), _rows(BN, 1),
          _full((2, 8)), _full((6, 8)),
          _full((23, 64)), _full((1, 64)), _full((64, 64)), _full((1, 64)),
          _full((64, 64)), _full((1, 64)), _full((64, 64)), _full((1, 64)),
      ],
      out_specs=[_rows(BN, 64), _rows(BN, 64)],
      out_shape=[jax.ShapeDtypeStruct((N, 64), _f32)] * 2,
  )(x, nt, sid, temb, semb, w1, b1, w2, b2, wl, bl, wr, br)


def _t2_body(pair, ea, we, attf, g, hbm, p, pay_o):
  pr = pair[...]
  xl = pr[:, 0:64]
  proj = ea[...] @ we[...]
  t = pr[:, 0:64] + pr[:, 64:128] + proj
  m = jnp.where(t > 0.0, t, 0.2 * t)
  ex = jnp.exp((m * attf[...]) @ g[...])
  w = xl * (ex @ hbm[...])
  e16 = ex @ p[...]
  z16 = jnp.zeros((BE, 16), _f32)
  pay_o[...] = jnp.concatenate(
      [w[:, 0:32], e16, z16, w[:, 32:64], e16, z16], axis=1)


def _t2(pair, ea, we, attf):
  rows = pair.shape[0]
  return pl.pallas_call(
      _t2_body,
      grid=(rows // BE,),
      in_specs=[
          _rows(BE, 128), _rows(BE, 7),
          _full((7, 64)), _full((1, 64)),
          _full((64, 4)), _full((4, 64)), _full((4, 16)),
      ],
      out_specs=_rows(BE, 128),
      out_shape=jax.ShapeDtypeStruct((rows, 128), _f32),
  )(pair, ea, we, attf, jnp.asarray(_G), jnp.asarray(_HB),
    jnp.asarray(_P))


def _t3_body(a01, a23, accex, q, bias, wl, bl, wr, br, xl_o, xr_o):
  den = accex[:, 0:4] @ q[...] + 1e-16
  accw = jnp.concatenate([a01[...], a23[...]], axis=1)
  g = accw / den + bias[...]
  h2 = jnp.maximum(g, 0.0)
  xl_o[...] = h2 @ wl[...] + bl[...]
  xr_o[...] = h2 @ wr[...] + br[...]


def _t3(a01, a23, accex, bias, wl, bl, wr, br):
  return pl.pallas_call(
      _t3_body,
      grid=(N // BN,),
      in_specs=[
          _rows(BN, 32), _rows(BN, 32), _rows(BN, 8), _full((4, 64)),
          _full((1, 64)),
          _full((64, 64)), _full((1, 64)), _full((64, 64)), _full((1, 64)),
      ],
      out_specs=[_rows(BN, 64), _rows(BN, 64)],
      out_shape=[jax.ShapeDtypeStruct((N, 64), _f32)] * 2,
  )(a01, a23, accex, jnp.asarray(_Q[:4]), bias, wl, bl, wr, br)


def _t4_body(a01, a23, accex, q, bias, wxr, wxz, wxn, brr, brz, bxn, bhn,
             lng, lnb, dw1, db1, dw2, db2, out_o, nh_o):
  den = accex[:, 0:4] @ q[...] + 1e-16
  accw = jnp.concatenate([a01[...], a23[...]], axis=1)
  hg = accw / den + bias[...]
  r = jax.nn.sigmoid(hg @ wxr[...] + brr[...])
  z = jax.nn.sigmoid(hg @ wxz[...] + brz[...])
  n = jnp.tanh(hg @ wxn[...] + bxn[...] + r * bhn[...])
  nh = (1.0 - z) * n
  mu = jnp.mean(nh, axis=1, keepdims=True)
  var = jnp.mean((nh - mu) ** 2, axis=1, keepdims=True)
  nh = (nh - mu) / jnp.sqrt(var + 1e-5) * lng[...] + lnb[...]
  nh_o[...] = nh
  out_o[...] = jnp.maximum(nh @ dw1[...] + db1[...], 0.0) @ dw2[...] + db2[...]


def _t4(a01, a23, accex, bias, wxr, wxz, wxn, brr, brz, bxn, bhn,
        lng, lnb, dw1, db1, dw2, db2):
  return pl.pallas_call(
      _t4_body,
      grid=(N // BN,),
      in_specs=[
          _rows(BN, 32), _rows(BN, 32), _rows(BN, 8), _full((4, 64)),
          _full((1, 64)),
          _full((64, 64)), _full((64, 64)), _full((64, 64)),
          _full((1, 64)), _full((1, 64)), _full((1, 64)), _full((1, 64)),
          _full((1, 64)), _full((1, 64)),
          _full((64, 64)), _full((1, 64)), _full((64, 7)), _full((1, 7)),
      ],
      out_specs=[_rows(BN, 7), _rows(BN, 64)],
      out_shape=[
          jax.ShapeDtypeStruct((N, 7), _f32),
          jax.ShapeDtypeStruct((N, 64), _f32),
      ],
  )(a01, a23, accex, jnp.asarray(_Q[:4]), bias, wxr, wxz, wxn, brr, brz, bxn,
    bhn, lng, lnb, dw1, db1, dw2, db2)


def kernel(x, node_type, sensor_id, edge_index, edge_attr, type_emb,
           sensor_emb, enc_W1, enc_b1, enc_W2, enc_b2,
           g1_Wl, g1_bl, g1_Wr, g1_br, g1_We, g1_att, g1_bias,
           g2_Wl, g2_bl, g2_Wr, g2_br, g2_We, g2_att, g2_bias,
           gru_Wx, gru_bx, gru_Wh, gru_bh, ln_g, ln_b,
           dec_W1, dec_b1, dec_W2, dec_b2):
  src = edge_index[0].astype(jnp.int32)
  dst = edge_index[1].astype(jnp.int32)
  pad = EPAD - E
  zi = jnp.zeros((pad,), jnp.int32)
  src_g = jnp.concatenate([src, zi])
  dst_g = jnp.concatenate([dst, zi])
  dst_s = jnp.concatenate(
      [dst, N + (jnp.arange(pad, dtype=jnp.int32) & 7)])
  dst2d = dst_s.reshape(EPAD // SUB, SUB)
  ea_pad = jnp.concatenate([edge_attr, jnp.zeros((pad, EDIM), _f32)], axis=0)
  z32 = jnp.zeros((ZR32, 32), _f32)

  nt2 = node_type.astype(jnp.int32).reshape(N, 1)
  sid2 = sensor_id.astype(jnp.int32).reshape(N, 1)

  def row(v):
    return v.reshape(1, -1)

  xl1, xr1 = _t1(x, nt2, sid2, type_emb, sensor_emb,
                 enc_W1, row(enc_b1), enc_W2, row(enc_b2),
                 g1_Wl, row(g1_bl), g1_Wr, row(g1_br))

  srcA, srcB = src_g[:EPADH], src_g[EPADH:]
  dstA, dstB = dst_g[:EPADH], dst_g[EPADH:]
  eaA, eaB = ea_pad[:EPADH], ea_pad[EPADH:]

  pr1a = _gather_pair(EPADH)(xl1, xr1, srcA, dstA)
  pr1b = _gather_pair(EPADH)(xl1, xr1, srcB, dstB)
  pay1a = _t2(pr1a, eaA, g1_We, row(g1_att))
  pay1b = _t2(pr1b, eaB, g1_We, row(g1_att))
  a01_1, a23_1, accex1 = _scatter()(pay1a, pay1b, dst2d, z32)

  xl2, xr2 = _t3(a01_1, a23_1, accex1, row(g1_bias),
                 g2_Wl, row(g2_bl), g2_Wr, row(g2_br))

  pr2a = _gather_pair(EPADH)(xl2, xr2, srcA, dstA)
  pr2b = _gather_pair(EPADH)(xl2, xr2, srcB, dstB)
  pay2a = _t2(pr2a, eaA, g2_We, row(g2_att))
  pay2b = _t2(pr2b, eaB, g2_We, row(g2_att))
  a01_2, a23_2, accex2 = _scatter()(pay2a, pay2b, dst2d, z32)

  denp = jnp.concatenate([accex2, jnp.zeros((N, 8), _f32)], axis=1)
  alpha2 = _alpha_kernel()(denp, dst_g, pay2a, pay2b)[:E, 0:4]

  out, new_hidden = _t4(
      a01_2, a23_2, accex2, row(g2_bias),
      gru_Wx[:, 0:64], gru_Wx[:, 64:128], gru_Wx[:, 128:192],
      row(gru_bx[0:64] + gru_bh[0:64]),
      row(gru_bx[64:128] + gru_bh[64:128]),
      row(gru_bx[128:192]), row(gru_bh[128:192]),
      row(ln_g), row(ln_b), dec_W1, row(dec_b1), dec_W2, row(dec_b2))

  return out, new_hidden, alpha2


# speedup vs baseline: 1.1877x; 1.1877x over previous
import functools

import numpy as np
import jax
import jax.numpy as jnp
from jax import lax
from jax.experimental import pallas as pl
from jax.experimental.pallas import tpu as pltpu
from jax.experimental.pallas import tpu_sc as plsc

N = 50000
E = 800000
HID = 64
H = 4
C = 16
EDIM = 7

NW = 32
SUB = 128
GROUP = 1024
EPAD = 819200
EPADH = EPAD // 2
NHALF = N // 2
ZROWS = 1568
ACC_ROWS = 16 * ZROWS
LAST_ROWS = NHALF - 15 * ZROWS

_f32 = jnp.float32

_G = np.kron(np.eye(H), np.ones((C, 1))).astype(np.float32)
_HB = np.kron(np.eye(H), np.ones((1, C))).astype(np.float32)
_P = np.eye(H, 16).astype(np.float32)
_Q = np.kron(np.eye(4, dtype=np.float32), np.ones((1, C), np.float32))
_Q = np.concatenate([_Q, np.zeros((12, 64), np.float32)], 0)

GG = 640


@functools.cache
def _gather_pair(rows):
  nper = rows // NW
  ngroups = nper // GG

  @functools.partial(
      pl.kernel,
      mesh=plsc.VectorSubcoreMesh(core_axis_name="c", subcore_axis_name="s"),
      compiler_params=pltpu.CompilerParams(use_tc_tiling_on_sc=False),
      out_type=jax.ShapeDtypeStruct((rows, 128), _f32),
      scratch_types=[
          pltpu.VMEM((2 * GG,), jnp.int32),
          pltpu.VMEM((GG, 64), _f32),
          pltpu.VMEM((GG, 64), _f32),
          pltpu.SemaphoreType.DMA,
          pltpu.SemaphoreType.DMA,
          pltpu.SemaphoreType.DMA,
      ],
  )
  def k(xl, xr, src, dst, out, idxv, bufl, bufr, seml, semr, wsem):
    wid = lax.axis_index("c") * 16 + lax.axis_index("s")
    base_w = wid * nper

    def body(g, carry):
      base = base_w + g * GG
      pltpu.sync_copy(src.at[pl.ds(base, GG)], idxv.at[pl.ds(0, GG)])
      pltpu.sync_copy(dst.at[pl.ds(base, GG)], idxv.at[pl.ds(GG, GG)])
      cpl = [
          pltpu.async_copy(xl.at[idxv.at[pl.ds(b * SUB, SUB)]],
                           bufl.at[pl.ds(b * SUB, SUB)], seml)
          for b in range(GG // SUB)
      ]
      cpr = [
          pltpu.async_copy(xr.at[idxv.at[pl.ds(GG + b * SUB, SUB)]],
                           bufr.at[pl.ds(b * SUB, SUB)], semr)
          for b in range(GG // SUB)
      ]
      for cp in cpl:
        cp.wait()
      w0 = pltpu.async_copy(bufl, out.at[pl.ds(base, GG), pl.ds(0, 64)], wsem)
      for cp in cpr:
        cp.wait()
      w1 = pltpu.async_copy(bufr, out.at[pl.ds(base, GG), pl.ds(64, 64)], wsem)
      w0.wait()
      w1.wait()
      return carry

    lax.fori_loop(0, ngroups, body, 0)

  return k


@functools.cache
def _alpha_kernel():
  nper = EPADH // NW
  ngroups = nper // GG

  @functools.partial(
      pl.kernel,
      mesh=plsc.VectorSubcoreMesh(core_axis_name="c", subcore_axis_name="s"),
      compiler_params=pltpu.CompilerParams(use_tc_tiling_on_sc=False),
      out_type=jax.ShapeDtypeStruct((EPAD, 16), _f32),
      scratch_types=[
          pltpu.VMEM((GG,), jnp.int32),
          pltpu.VMEM((GG, 16), _f32),
          pltpu.VMEM((GG, 16), _f32),
          pltpu.SemaphoreType.DMA,
      ],
  )
  def k(den, dst, paya, payb, out, idxv, dbuf, ebuf, sem):
    wid = lax.axis_index("c") * 16 + lax.axis_index("s")

    for h, pay in enumerate((paya, payb)):
      def body(g, carry, pay=pay, h=h):
        lbase = wid * nper + g * GG
        base = h * EPADH + lbase
        pltpu.sync_copy(dst.at[pl.ds(base, GG)], idxv)
        cps = [
            pltpu.async_copy(den.at[idxv.at[pl.ds(b * SUB, SUB)]],
                             dbuf.at[pl.ds(b * SUB, SUB)], sem)
            for b in range(GG // SUB)
        ]
        pltpu.sync_copy(pay.at[pl.ds(lbase, GG), pl.ds(64, 16)], ebuf)
        for cp in cps:
          cp.wait()

        def div_row(r, carry2):
          ebuf[r] = ebuf[r] / (dbuf[r] + 1e-16)
          return carry2

        lax.fori_loop(0, GG, div_row, 0)
        pltpu.sync_copy(ebuf, out.at[pl.ds(base, GG)])
        return carry

      lax.fori_loop(0, ngroups, body, 0)

  return k


ZR32 = 3128
ACC32 = 16 * ZR32
LAST32 = N - 15 * ZR32


@functools.cache
def _scatter32():
  group = 512
  nper = EPADH // 16
  ngroups = nper // group

  @functools.partial(
      pl.kernel,
      mesh=plsc.VectorSubcoreMesh(core_axis_name="c", subcore_axis_name="s"),
      compiler_params=pltpu.CompilerParams(use_tc_tiling_on_sc=False),
      out_type=[
          jax.ShapeDtypeStruct((N, 32), _f32),
          jax.ShapeDtypeStruct((N, 32), _f32),
      ],
      scratch_types=[
          pltpu.VMEM_SHARED((ACC32, 32), _f32),
          pltpu.VMEM((group, 32), _f32),
          pltpu.VMEM((group // SUB, SUB), jnp.int32),
          pltpu.SemaphoreType.DMA,
          pltpu.SemaphoreType.DMA,
      ],
  )
  def k(paya, payb, dst2d, z32, out01, out23, acc, wbuf, idx2, semi, semp):
    c = lax.axis_index("c")
    s = lax.axis_index("s")
    pltpu.sync_copy(z32, acc.at[pl.ds(s * ZR32, ZR32)])
    plsc.subcore_barrier()

    def run(col0):
      for h, pay in enumerate((paya, payb)):
        def body(g, carry, pay=pay, h=h):
          base = s * nper + g * group
          grow = (h * EPADH + s * nper + g * group) // SUB
          cpi = pltpu.async_copy(dst2d.at[pl.ds(grow, group // SUB)], idx2,
                                 semi)
          cpp = pltpu.async_copy(pay.at[pl.ds(base, group), pl.ds(col0, 32)],
                                 wbuf, semp)
          cpi.wait()
          cpp.wait()
          for b in range(group // SUB):
            pltpu.sync_copy(wbuf.at[pl.ds(b * SUB, SUB)], acc.at[idx2.at[b]],
                            add=True)
          return carry

        lax.fori_loop(0, ngroups, body, 0)

    @pl.when(c == 0)
    def _():
      run(0)

    @pl.when(c == 1)
    def _():
      run(32)

    plsc.subcore_barrier()

    def dump(out_ref):
      @pl.when(s < 15)
      def _():
        pltpu.sync_copy(acc.at[pl.ds(s * ZR32, ZR32)],
                        out_ref.at[pl.ds(s * ZR32, ZR32)])

      @pl.when(s == 15)
      def _():
        pltpu.sync_copy(acc.at[pl.ds(15 * ZR32, LAST32)],
                        out_ref.at[pl.ds(15 * ZR32, LAST32)])

    @pl.when(c == 0)
    def _():
      dump(out01)

    @pl.when(c == 1)
    def _():
      dump(out23)

  return k


def _make_scatter(width, group):
  nper = EPADH // 16
  ngroups = nper // group

  @functools.partial(
      pl.kernel,
      mesh=plsc.VectorSubcoreMesh(core_axis_name="c", subcore_axis_name="s"),
      compiler_params=pltpu.CompilerParams(use_tc_tiling_on_sc=False),
      out_type=jax.ShapeDtypeStruct((N, width), _f32),
      scratch_types=[
          pltpu.VMEM_SHARED((ACC_ROWS, width), _f32),
          pltpu.VMEM((group, width), _f32),
          pltpu.VMEM((group // SUB, SUB), jnp.int32),
          pltpu.SemaphoreType.DMA,
      ],
  )
  def k(paya, payb, dst2d, zrows, acc_o, acc, wbuf, idx2, semp):
    c = lax.axis_index("c")
    s = lax.axis_index("s")
    pltpu.sync_copy(zrows, acc.at[pl.ds(s * ZROWS, ZROWS)])
    plsc.subcore_barrier()

    nb = c * NHALF

    for h, pay in enumerate((paya, payb)):
      def body(g, carry, pay=pay, h=h):
        base = s * nper + g * group
        grow = (h * EPADH + s * nper + g * group) // SUB
        cpp = pltpu.async_copy(pay.at[pl.ds(base, group), pl.ds(64, width)],
                               wbuf, semp)
        pltpu.sync_copy(dst2d.at[pl.ds(grow, group // SUB)], idx2)
        for r in range(group // SUB):
          for q in range(SUB // 16):
            v = idx2[r, pl.ds(q * 16, 16)]
            local = v - nb
            ok = (local >= 0) & (local < NHALF)
            dump = NHALF + (lax.iota(jnp.int32, 16) & 7)
            idx2[r, pl.ds(q * 16, 16)] = jnp.where(ok, local, dump)
        cpp.wait()
        for b in range(group // SUB):
          pltpu.sync_copy(wbuf.at[pl.ds(b * SUB, SUB)], acc.at[idx2.at[b]],
                          add=True)
        return carry

      lax.fori_loop(0, ngroups, body, 0)
    plsc.subcore_barrier()

    hb = c * NHALF

    @pl.when(s < 15)
    def _():
      pltpu.sync_copy(acc.at[pl.ds(s * ZROWS, ZROWS)],
                      acc_o.at[pl.ds(hb + s * ZROWS, ZROWS)])

    @pl.when(s == 15)
    def _():
      pltpu.sync_copy(acc.at[pl.ds(15 * ZROWS, LAST_ROWS)],
                      acc_o.at[pl.ds(hb + 15 * ZROWS, LAST_ROWS)])

  return k


@functools.cache
def _scatter16():
  return _make_scatter(16, 1024)


BN = 2000
BE = 8192
BE5 = 8000


def _full(shape):
  return pl.BlockSpec(shape, lambda i: tuple(0 for _ in shape))


def _rows(block, width):
  return pl.BlockSpec((block, width), lambda i: (i, 0))


def _t1_body(x, nt, sid, temb, semb, w1, b1, w2, b2, wl, bl, wr, br,
             xl_o, xr_o):
  xx = x[...]
  oh_t = (nt[...] == lax.broadcasted_iota(jnp.int32, (BN, 2), 1)).astype(_f32)
  oh_s = (sid[...] == lax.broadcasted_iota(jnp.int32, (BN, 6), 1)).astype(_f32)
  hcat = jnp.concatenate([xx, oh_t @ temb[...], oh_s @ semb[...]], axis=1)
  h = jnp.maximum(hcat @ w1[...] + b1[...], 0.0) @ w2[...] + b2[...]
  xl_o[...] = h @ wl[...] + bl[...]
  xr_o[...] = h @ wr[...] + br[...]


def _t1(x, nt, sid, temb, semb, w1, b1, w2, b2, wl, bl, wr, br):
  return pl.pallas_call(
      _t1_body,
      grid=(N // BN,),
      in_specs=[
          _rows(BN, 7), _rows(BN, 1), _rows(BN, 1),
          _full((2, 8)), _full((6, 8)),
          _full((23, 64)), _full((1, 64)), _full((64, 64)), _full((1, 64)),
          _full((64, 64)), _full((1, 64)), _full((64, 64)), _full((1, 64)),
      ],
      out_specs=[_rows(BN, 64), _rows(BN, 64)],
      out_shape=[jax.ShapeDtypeStruct((N, 64), _f32)] * 2,
  )(x, nt, sid, temb, semb, w1, b1, w2, b2, wl, bl, wr, br)


def _t2_body(pair, ea, we, attf, g, hbm, p, pay_o):
  pr = pair[...]
  xl = pr[:, 0:64]
  proj = ea[...] @ we[...]
  t = pr[:, 0:64] + pr[:, 64:128] + proj
  m = jnp.where(t > 0.0, t, 0.2 * t)
  ex = jnp.exp((m * attf[...]) @ g[...])
  w = xl * (ex @ hbm[...])
  pay_o[...] = jnp.concatenate(
      [w, ex @ p[...], jnp.zeros((BE, 48), _f32)], axis=1)


def _t2(pair, ea, we, attf):
  rows = pair.shape[0]
  return pl.pallas_call(
      _t2_body,
      grid=(rows // BE,),
      in_specs=[
          _rows(BE, 128), _rows(BE, 7),
          _full((7, 64)), _full((1, 64)),
          _full((64, 4)), _full((4, 64)), _full((4, 16)),
      ],
      out_specs=_rows(BE, 128),
      out_shape=jax.ShapeDtypeStruct((rows, 128), _f32),
  )(pair, ea, we, attf, jnp.asarray(_G), jnp.asarray(_HB),
    jnp.asarray(_P))


def _t3_body(a01, a23, accex, q, bias, wl, bl, wr, br, xl_o, xr_o):
  den = accex[...] @ q[...] + 1e-16
  accw = jnp.concatenate([a01[...], a23[...]], axis=1)
  g = accw / den + bias[...]
  h2 = jnp.maximum(g, 0.0)
  xl_o[...] = h2 @ wl[...] + bl[...]
  xr_o[...] = h2 @ wr[...] + br[...]


def _t3(a01, a23, accex, bias, wl, bl, wr, br):
  return pl.pallas_call(
      _t3_body,
      grid=(N // BN,),
      in_specs=[
          _rows(BN, 32), _rows(BN, 32), _rows(BN, 16), _full((16, 64)),
          _full((1, 64)),
          _full((64, 64)), _full((1, 64)), _full((64, 64)), _full((1, 64)),
      ],
      out_specs=[_rows(BN, 64), _rows(BN, 64)],
      out_shape=[jax.ShapeDtypeStruct((N, 64), _f32)] * 2,
  )(a01, a23, accex, jnp.asarray(_Q), bias, wl, bl, wr, br)


def _t4_body(a01, a23, accex, q, bias, wxr, wxz, wxn, brr, brz, bxn, bhn,
             lng, lnb, dw1, db1, dw2, db2, out_o, nh_o):
  den = accex[...] @ q[...] + 1e-16
  accw = jnp.concatenate([a01[...], a23[...]], axis=1)
  hg = accw / den + bias[...]
  r = jax.nn.sigmoid(hg @ wxr[...] + brr[...])
  z = jax.nn.sigmoid(hg @ wxz[...] + brz[...])
  n = jnp.tanh(hg @ wxn[...] + bxn[...] + r * bhn[...])
  nh = (1.0 - z) * n
  mu = jnp.mean(nh, axis=1, keepdims=True)
  var = jnp.mean((nh - mu) ** 2, axis=1, keepdims=True)
  nh = (nh - mu) / jnp.sqrt(var + 1e-5) * lng[...] + lnb[...]
  nh_o[...] = nh
  out_o[...] = jnp.maximum(nh @ dw1[...] + db1[...], 0.0) @ dw2[...] + db2[...]


def _t4(a01, a23, accex, bias, wxr, wxz, wxn, brr, brz, bxn, bhn,
        lng, lnb, dw1, db1, dw2, db2):
  return pl.pallas_call(
      _t4_body,
      grid=(N // BN,),
      in_specs=[
          _rows(BN, 32), _rows(BN, 32), _rows(BN, 16), _full((16, 64)),
          _full((1, 64)),
          _full((64, 64)), _full((64, 64)), _full((64, 64)),
          _full((1, 64)), _full((1, 64)), _full((1, 64)), _full((1, 64)),
          _full((1, 64)), _full((1, 64)),
          _full((64, 64)), _full((1, 64)), _full((64, 7)), _full((1, 7)),
      ],
      out_specs=[_rows(BN, 7), _rows(BN, 64)],
      out_shape=[
          jax.ShapeDtypeStruct((N, 7), _f32),
          jax.ShapeDtypeStruct((N, 64), _f32),
      ],
  )(a01, a23, accex, jnp.asarray(_Q), bias, wxr, wxz, wxn, brr, brz, bxn,
    bhn, lng, lnb, dw1, db1, dw2, db2)


def kernel(x, node_type, sensor_id, edge_index, edge_attr, type_emb,
           sensor_emb, enc_W1, enc_b1, enc_W2, enc_b2,
           g1_Wl, g1_bl, g1_Wr, g1_br, g1_We, g1_att, g1_bias,
           g2_Wl, g2_bl, g2_Wr, g2_br, g2_We, g2_att, g2_bias,
           gru_Wx, gru_bx, gru_Wh, gru_bh, ln_g, ln_b,
           dec_W1, dec_b1, dec_W2, dec_b2):
  src = edge_index[0].astype(jnp.int32)
  dst = edge_index[1].astype(jnp.int32)
  pad = EPAD - E
  zi = jnp.zeros((pad,), jnp.int32)
  src_g = jnp.concatenate([src, zi])
  dst_g = jnp.concatenate([dst, zi])
  dst_s = jnp.concatenate(
      [dst, N + (jnp.arange(pad, dtype=jnp.int32) & 7)])
  dst2d = dst_s.reshape(EPAD // SUB, SUB)
  ea_pad = jnp.concatenate([edge_attr, jnp.zeros((pad, EDIM), _f32)], axis=0)
  z32 = jnp.zeros((ZR32, 32), _f32)
  z16 = jnp.zeros((ZROWS, 16), _f32)

  nt2 = node_type.astype(jnp.int32).reshape(N, 1)
  sid2 = sensor_id.astype(jnp.int32).reshape(N, 1)

  def row(v):
    return v.reshape(1, -1)

  xl1, xr1 = _t1(x, nt2, sid2, type_emb, sensor_emb,
                 enc_W1, row(enc_b1), enc_W2, row(enc_b2),
                 g1_Wl, row(g1_bl), g1_Wr, row(g1_br))

  srcA, srcB = src_g[:EPADH], src_g[EPADH:]
  dstA, dstB = dst_g[:EPADH], dst_g[EPADH:]
  eaA, eaB = ea_pad[:EPADH], ea_pad[EPADH:]

  pr1a = _gather_pair(EPADH)(xl1, xr1, srcA, dstA)
  pr1b = _gather_pair(EPADH)(xl1, xr1, srcB, dstB)
  pay1a = _t2(pr1a, eaA, g1_We, row(g1_att))
  pay1b = _t2(pr1b, eaB, g1_We, row(g1_att))
  a01_1, a23_1 = _scatter32()(pay1a, pay1b, dst2d, z32)
  accex1 = _scatter16()(pay1a, pay1b, dst2d, z16)

  xl2, xr2 = _t3(a01_1, a23_1, accex1, row(g1_bias),
                 g2_Wl, row(g2_bl), g2_Wr, row(g2_br))

  pr2a = _gather_pair(EPADH)(xl2, xr2, srcA, dstA)
  pr2b = _gather_pair(EPADH)(xl2, xr2, srcB, dstB)
  pay2a = _t2(pr2a, eaA, g2_We, row(g2_att))
  pay2b = _t2(pr2b, eaB, g2_We, row(g2_att))
  a01_2, a23_2 = _scatter32()(pay2a, pay2b, dst2d, z32)
  accex2 = _scatter16()(pay2a, pay2b, dst2d, z16)

  alpha2 = _alpha_kernel()(accex2, dst_g, pay2a, pay2b)[:E, 0:4]

  out, new_hidden = _t4(
      a01_2, a23_2, accex2, row(g2_bias),
      gru_Wx[:, 0:64], gru_Wx[:, 64:128], gru_Wx[:, 128:192],
      row(gru_bx[0:64] + gru_bh[0:64]),
      row(gru_bx[64:128] + gru_bh[64:128]),
      row(gru_bx[128:192]), row(gru_bh[128:192]),
      row(ln_g), row(ln_b), dec_W1, row(dec_b1), dec_W2, row(dec_b2))

  return out, new_hidden, alpha2
